# Initial kernel scaffold; baseline (speedup 1.0000x reference)
import jax
import jax.numpy as jnp
from jax.experimental import pallas as pl


def kernel(p_node_feat, p_edge_index, r_node_feat, r_edge_index, Wp1, bp1, Wp2, bp2, Wr1, br1, Wr2, br2, Wm1, bm1, Wm2, bm2, Wm3, bm3):
    raise NotImplementedError("write your pallas kernel here")


# speedup vs baseline: 13.9235x; 13.9235x over previous
import functools

import jax
import jax.numpy as jnp
from jax import lax
from jax.experimental import pallas as pl
from jax.experimental.pallas import tpu as pltpu
from jax.experimental.pallas import tpu_sc as plsc

N = 10000
E = 320000
D = 128
NC = 2
NS = 16
K = 128
CHUNKS = E // K
ROWS_PER_TILE = (N // NS) // 8 * 8
ROWS_TAIL = N - NS * ROWS_PER_TILE
NPAD = ((N + NS * 16 - 1) // (NS * 16)) * NS * 16
DEG_PER_TILE = NPAD // NS

def _deg_body(dst_hbm, out_hbm, acc, zbuf, ones_v, didx_v):
    c = lax.axis_index("c")
    s = lax.axis_index("s")

    def fill(i, _):
        zbuf[pl.ds(i * 16, 16)] = jnp.zeros((16,), jnp.float32)
        ones_v[pl.ds((i % 8) * 16, 16)] = jnp.ones((16,), jnp.float32)
        return 0

    lax.fori_loop(0, DEG_PER_TILE // 16, fill, 0)
    pltpu.sync_copy(zbuf, acc.at[pl.ds(s * DEG_PER_TILE, DEG_PER_TILE)])
    plsc.subcore_barrier()

    def body(i, _):
        cid = s + i * NS

        @pl.when(cid < CHUNKS)
        def _():
            goff = pl.multiple_of(c * E + cid * K, K)
            pltpu.sync_copy(dst_hbm.at[pl.ds(goff, K)], didx_v)
            pltpu.sync_copy(ones_v, acc.at[didx_v], add=True)

        return 0

    lax.fori_loop(0, (CHUNKS + NS - 1) // NS, body, 0)
    plsc.subcore_barrier()
    ooff = pl.multiple_of(c * NPAD + s * DEG_PER_TILE, 8)
    pltpu.sync_copy(acc.at[pl.ds(s * DEG_PER_TILE, DEG_PER_TILE)],
                    out_hbm.at[pl.ds(ooff, DEG_PER_TILE)])


def _edge_body(y_hbm, src_hbm, dst_hbm, out_hbm, acc, sidx_v, didx_v,
               rows_v, sem):
    c = lax.axis_index("c")
    s = lax.axis_index("s")

    roff = pl.multiple_of(s * ROWS_PER_TILE, 8)
    groff = pl.multiple_of(c * N + s * ROWS_PER_TILE, 8)
    pltpu.sync_copy(y_hbm.at[pl.ds(groff, ROWS_PER_TILE)],
                    acc.at[pl.ds(roff, ROWS_PER_TILE)])
    tail = NS * ROWS_PER_TILE

    @pl.when(s == 0)
    def _():
        pltpu.sync_copy(y_hbm.at[pl.ds(pl.multiple_of(c * N + tail, 8),
                                       ROWS_TAIL)],
                        acc.at[pl.ds(tail, ROWS_TAIL)])

    plsc.subcore_barrier()

    def body(i, _):
        cid = s + i * NS

        @pl.when(cid < CHUNKS)
        def _():
            goff = pl.multiple_of(c * E + cid * K, K)
            pltpu.sync_copy(src_hbm.at[pl.ds(goff, K)], sidx_v)
            pltpu.sync_copy(dst_hbm.at[pl.ds(goff, K)], didx_v)
            pltpu.async_copy(y_hbm.at[sidx_v], rows_v, sem).wait()
            pltpu.sync_copy(rows_v, acc.at[didx_v], add=True)

        return 0

    lax.fori_loop(0, (CHUNKS + NS - 1) // NS, body, 0)
    plsc.subcore_barrier()
    pltpu.sync_copy(acc.at[pl.ds(roff, ROWS_PER_TILE)],
                    out_hbm.at[pl.ds(groff, ROWS_PER_TILE)])

    @pl.when(s == 0)
    def _():
        pltpu.sync_copy(acc.at[pl.ds(tail, ROWS_TAIL)],
                        out_hbm.at[pl.ds(pl.multiple_of(c * N + tail, 8),
                                         ROWS_TAIL)])


@functools.lru_cache(maxsize=None)
def _sc_kernels():
    mesh = plsc.VectorSubcoreMesh(core_axis_name="c", subcore_axis_name="s",
                                  num_cores=NC, num_subcores=NS)
    deg_k = pl.kernel(
        _deg_body,
        mesh=mesh,
        out_type=jax.ShapeDtypeStruct((2 * NPAD,), jnp.float32),
        scratch_types=[
            pltpu.VMEM_SHARED((NPAD,), jnp.float32),
            pltpu.VMEM((DEG_PER_TILE,), jnp.float32),
            pltpu.VMEM((K,), jnp.float32),
            pltpu.VMEM((K,), jnp.int32),
        ],
    )
    edge_k = pl.kernel(
        _edge_body,
        mesh=mesh,
        out_type=jax.ShapeDtypeStruct((2 * N, D), jnp.float32),
        scratch_types=[
            pltpu.VMEM_SHARED((N, D), jnp.float32),
            pltpu.VMEM((K,), jnp.int32),
            pltpu.VMEM((K,), jnp.int32),
            pltpu.VMEM((K, D), jnp.float32),
            pltpu.SemaphoreType.DMA,
        ],
    )
    return deg_k, edge_k


_RB = 1000
_GRID = 2 * N // _RB


def _scale_body(x_ref, deg_ref, o_ref):
    dinv = lax.rsqrt(deg_ref[...] + 1.0)
    o_ref[...] = x_ref[...] * dinv


def _scale(x, deg):
    return pl.pallas_call(
        _scale_body,
        grid=(_GRID,),
        in_specs=[
            pl.BlockSpec((_RB, D), lambda i: (i, 0)),
            pl.BlockSpec((_RB, 1), lambda i: (i, 0)),
        ],
        out_specs=pl.BlockSpec((_RB, D), lambda i: (i, 0)),
        out_shape=jax.ShapeDtypeStruct((2 * N, D), jnp.float32),
    )(x, deg)


def _conv_relu_body(z_ref, deg_ref, w_ref, b_ref, o_ref):
    dinv = lax.rsqrt(deg_ref[...] + 1.0)
    h = jnp.dot(z_ref[...] * dinv, w_ref[0],
                preferred_element_type=jnp.float32) + b_ref[0]
    o_ref[...] = jnp.maximum(h, 0.0) * dinv


def _conv_relu_scale(z, deg, w2, b2):
    return pl.pallas_call(
        _conv_relu_body,
        grid=(_GRID,),
        in_specs=[
            pl.BlockSpec((_RB, D), lambda i: (i, 0)),
            pl.BlockSpec((_RB, 1), lambda i: (i, 0)),
            pl.BlockSpec((1, D, D), lambda i: (i // (_GRID // 2), 0, 0)),
            pl.BlockSpec((1, 1, D), lambda i: (i // (_GRID // 2), 0, 0)),
        ],
        out_specs=pl.BlockSpec((_RB, D), lambda i: (i, 0)),
        out_shape=jax.ShapeDtypeStruct((2 * N, D), jnp.float32),
    )(z, deg, w2, b2)


def _head_body(z_ref, deg_ref, w_ref, b_ref, wm1, bm1, wm2, bm2, wm3, bm3,
               o_ref):
    dinv = lax.rsqrt(deg_ref[...] + 1.0)
    emb = jnp.dot(z_ref[...] * dinv, w_ref[0],
                  preferred_element_type=jnp.float32) + b_ref[0]
    h = jnp.maximum(jnp.dot(emb, wm1[...],
                            preferred_element_type=jnp.float32) + bm1[...], 0.0)
    h = jnp.maximum(jnp.dot(h, wm2[...],
                            preferred_element_type=jnp.float32) + bm2[...], 0.0)
    o_ref[...] = jnp.dot(h, wm3[...],
                         preferred_element_type=jnp.float32) + bm3[...]


def _head(z, deg, w2, b2, wm1, bm1, wm2, bm2, wm3, bm3):
    full = lambda shape: pl.BlockSpec(shape, lambda i: (0,) * len(shape))
    return pl.pallas_call(
        _head_body,
        grid=(_GRID,),
        in_specs=[
            pl.BlockSpec((_RB, D), lambda i: (i, 0)),
            pl.BlockSpec((_RB, 1), lambda i: (i, 0)),
            pl.BlockSpec((1, D, D), lambda i: (i // (_GRID // 2), 0, 0)),
            pl.BlockSpec((1, 1, D), lambda i: (i // (_GRID // 2), 0, 0)),
            full((D, D)), full((1, D)), full((D, D)), full((1, D)),
            full((D, 1)), full((1, 1)),
        ],
        out_specs=pl.BlockSpec((_RB, 1), lambda i: (i, 0)),
        out_shape=jax.ShapeDtypeStruct((2 * N, 1), jnp.float32),
    )(z, deg, w2, b2, wm1, bm1, wm2, bm2, wm3, bm3)


def kernel(p_node_feat, p_edge_index, r_node_feat, r_edge_index,
           Wp1, bp1, Wp2, bp2, Wr1, br1, Wr2, br2,
           Wm1, bm1, Wm2, bm2, Wm3, bm3):
    x = jnp.concatenate([p_node_feat, r_node_feat], axis=0)
    src2 = jnp.concatenate([p_edge_index[0], r_edge_index[0] + N])
    dst2 = jnp.concatenate([p_edge_index[1], r_edge_index[1]])

    deg_kernel, edge_kernel = _sc_kernels()
    deg_raw = deg_kernel(dst2)
    deg = jnp.concatenate([deg_raw[:N], deg_raw[NPAD:NPAD + N]])
    deg = deg.reshape(2 * N, 1)

    w1 = jnp.stack([Wp1, Wr1])
    b1 = jnp.stack([bp1, br1]).reshape(2, 1, D)
    w2 = jnp.stack([Wp2, Wr2])
    b2 = jnp.stack([bp2, br2]).reshape(2, 1, D)

    y1 = _scale(x, deg)
    z1 = edge_kernel(y1, src2, dst2)
    y2 = _conv_relu_scale(z1, deg, w1, b1)
    z2 = edge_kernel(y2, src2, dst2)
    return _head(z2, deg, w2, b2,
                 Wm1, bm1.reshape(1, D), Wm2, bm2.reshape(1, D),
                 Wm3, bm3.reshape(1, 1))
